# Initial kernel scaffold; baseline (speedup 1.0000x reference)
import jax
import jax.numpy as jnp
from jax.experimental import pallas as pl


def kernel(ufeat, ifeat, edge_src, edge_dst, edge_rating, implicit_matrix, sqrt_counts, global_mean, W_u, W_i, W_uo, b_uo, W_io, b_io, Bu, Bi, Y_table):
    raise NotImplementedError("write your pallas kernel here")


# speedup vs baseline: 2.2980x; 2.2980x over previous
import functools

import jax
import jax.numpy as jnp
from jax import lax
from jax.experimental import pallas as pl
from jax.experimental.pallas import tpu as pltpu
from jax.experimental.pallas import tpu_sc as plsc

N_U, N_I = 10000, 1000
D_IN, D_AGG, D_OUT = 256, 256, 64
N_R = 5
E = 160000
L_IMP = 50

NC, NS = 2, 16
E_PAD = 163840
EPT = E_PAD // NS
N_UP = 10240
UPW = N_UP // (NC * NS)
DUMMY_U = 10200
DUMMY_I = 1020
DUMMY_B = 5020

_mesh = plsc.VectorSubcoreMesh(core_axis_name="c", subcore_axis_name="s")


def _fill2d(ref, nrows, ncols, value):
    v = jnp.full((16,), value, jnp.float32)

    def body(j, _):
        for k in range(ncols // 16):
            ref[j, pl.ds(k * 16, 16)] = v
        return 0

    lax.fori_loop(0, nrows, body, 0)


@functools.partial(
    pl.kernel,
    out_type=(
        jax.ShapeDtypeStruct((1280, 128), jnp.float32),
        jax.ShapeDtypeStruct((128, 128), jnp.float32),
        jax.ShapeDtypeStruct((N_UP, 128), jnp.float32),
    ),
    mesh=_mesh,
    scratch_types=[
        pltpu.VMEM((80, 128), jnp.int32),
        pltpu.VMEM((80, 128), jnp.int32),
        pltpu.VMEM((125, 128), jnp.int32),
        pltpu.VMEM((125, 128), jnp.int32),
        pltpu.VMEM((128, 128), jnp.float32),
        pltpu.VMEM((64, 128), jnp.float32),
        pltpu.VMEM_SHARED((1280, 128), jnp.float32),
        pltpu.VMEM_SHARED((N_UP // 2, 128), jnp.float32),
    ],
)
def _sc_pre(degg_hbm, degs_hbm, onest_hbm, impg_hbm, impu_hbm, yt_hbm,
            degu_out, degi_out, y_out,
            dgi_v, dsi_v, gidx_v, uidx_v, rows_v, zb_v,
            deg_sh, y_sh):
    c = lax.axis_index("c")
    s = lax.axis_index("s")
    w = c * NS + s
    _fill2d(zb_v, 64, 128, 0.0)
    pltpu.sync_copy(zb_v, deg_sh.at[pl.ds(s * 80, 64)])
    pltpu.sync_copy(zb_v.at[pl.ds(0, 16)], deg_sh.at[pl.ds(s * 80 + 64, 16)])
    for k in range(5):
        pltpu.sync_copy(zb_v, y_sh.at[pl.ds(s * 320 + k * 64, 64)])
    plsc.subcore_barrier()

    pltpu.sync_copy(degg_hbm.at[c, s], dgi_v)
    pltpu.sync_copy(degs_hbm.at[c, s], dsi_v)

    def deg_body(j, _):
        pltpu.sync_copy(onest_hbm.at[dgi_v.at[j]], rows_v)
        pltpu.sync_copy(rows_v, deg_sh.at[dsi_v.at[j]], add=True)
        return 0

    lax.fori_loop(0, 80, deg_body, 0)

    pltpu.sync_copy(impg_hbm.at[w], gidx_v)
    pltpu.sync_copy(impu_hbm.at[w], uidx_v)

    def y_body(j, _):
        pltpu.sync_copy(yt_hbm.at[gidx_v.at[j]], rows_v)
        pltpu.sync_copy(rows_v, y_sh.at[uidx_v.at[j]], add=True)
        return 0

    lax.fori_loop(0, 125, y_body, 0)
    plsc.subcore_barrier()

    @pl.when(c == 0)
    def _():
        pltpu.sync_copy(deg_sh.at[pl.ds(s * 80, 80)],
                        degu_out.at[pl.ds(s * 80, 80)])

    @pl.when(c == 1)
    def _():
        pltpu.sync_copy(deg_sh.at[pl.ds(s * 8, 8)],
                        degi_out.at[pl.ds(s * 8, 8)])

    pltpu.sync_copy(y_sh.at[pl.ds(s * 320, 320)],
                    y_out.at[pl.ds(c * 5120 + s * 320, 320)])


@functools.partial(
    pl.kernel,
    out_type=(
        jax.ShapeDtypeStruct((N_UP, 128), jnp.float32),
        jax.ShapeDtypeStruct((2 * N_UP, 128), jnp.float32),
    ),
    mesh=_mesh,
    scratch_types=[
        pltpu.VMEM((80, 128), jnp.int32),
        pltpu.VMEM((80, 128), jnp.int32),
        pltpu.VMEM((128, 128), jnp.float32),
        pltpu.VMEM((64, 128), jnp.float32),
        pltpu.VMEM_SHARED((N_UP, 128), jnp.float32),
    ],
)
def _sc_edge(ufs2_hbm, msA_hbm, msB_hbm, g0_hbm, s0_hbm, g1_hbm, s1_hbm,
             a_out, s_out,
             gi_v, si_v, rows_v, zb_v, acc_sh):
    c = lax.axis_index("c")
    s = lax.axis_index("s")
    _fill2d(zb_v, 64, 128, 0.0)

    def zero_acc():
        for k in range(10):
            pltpu.sync_copy(zb_v, acc_sh.at[pl.ds(s * 640 + k * 64, 64)])

    zero_acc()
    plsc.subcore_barrier()

    @pl.when(c == 0)
    def _():
        for h in range(2):
            pltpu.sync_copy(g0_hbm.at[s, pl.ds(h * 80, 80)], gi_v)
            pltpu.sync_copy(s0_hbm.at[s, pl.ds(h * 80, 80)], si_v)

            def body(j, _):
                pltpu.sync_copy(ufs2_hbm.at[gi_v.at[j]], rows_v)
                pltpu.sync_copy(rows_v, acc_sh.at[si_v.at[j]], add=True)
                return 0

            lax.fori_loop(0, 80, body, 0)
        plsc.subcore_barrier()
        pltpu.sync_copy(acc_sh.at[pl.ds(s * 640, 640)],
                        a_out.at[pl.ds(s * 640, 640)])

    @pl.when(c == 1)
    def _():
        pltpu.sync_copy(g1_hbm.at[s], gi_v)
        pltpu.sync_copy(s1_hbm.at[s], si_v)

        def bodyA(j, _):
            pltpu.sync_copy(msA_hbm.at[gi_v.at[j]], rows_v)
            pltpu.sync_copy(rows_v, acc_sh.at[si_v.at[j]], add=True)
            return 0

        lax.fori_loop(0, 80, bodyA, 0)
        plsc.subcore_barrier()
        pltpu.sync_copy(acc_sh.at[pl.ds(s * 640, 640)],
                        s_out.at[pl.ds(s * 640, 640)])
        zero_acc()
        plsc.subcore_barrier()

        def bodyB(j, _):
            pltpu.sync_copy(msB_hbm.at[gi_v.at[j]], rows_v)
            pltpu.sync_copy(rows_v, acc_sh.at[si_v.at[j]], add=True)
            return 0

        lax.fori_loop(0, 80, bodyB, 0)
        plsc.subcore_barrier()
        pltpu.sync_copy(acc_sh.at[pl.ds(s * 640, 640)],
                        s_out.at[pl.ds(N_UP + s * 640, 640)])


def _interleave2(idx):
    return jnp.stack([2 * idx, 2 * idx + 1], axis=-1).reshape(-1)


def kernel(ufeat, ifeat, edge_src, edge_dst, edge_rating, implicit_matrix,
           sqrt_counts, global_mean, W_u, W_i, W_uo, b_uo, W_io, b_io,
           Bu, Bi, Y_table):
    pad = E_PAD - E
    src_p = jnp.concatenate([edge_src, jnp.full((pad,), DUMMY_U, jnp.int32)])
    dst_p = jnp.concatenate([edge_dst, jnp.full((pad,), DUMMY_I, jnp.int32)])
    idxb = edge_rating * N_I + edge_dst
    idxb_p = jnp.concatenate([idxb, jnp.full((pad,), DUMMY_B, jnp.int32)])
    src_g = jnp.concatenate([edge_src, jnp.zeros((pad,), jnp.int32)])
    idxb_g = jnp.concatenate([idxb, jnp.zeros((pad,), jnp.int32)])

    degg = jnp.stack([src_p % 8, dst_p % 8]).reshape(NC, NS, 80, 128)
    degs = jnp.stack([src_p // 8, dst_p // 8]).reshape(NC, NS, 80, 128)
    onest = (jnp.arange(128, dtype=jnp.int32) // 16
             == jnp.arange(8, dtype=jnp.int32)[:, None]).astype(jnp.float32)
    g0 = _interleave2(src_g).reshape(NS, 160, 128)
    s0 = _interleave2(idxb_p).reshape(NS, 160, 128)
    g1 = idxb_g.reshape(NS, 80, 128)
    s1 = src_p.reshape(NS, 80, 128)

    imp_p = jnp.concatenate(
        [implicit_matrix, jnp.zeros((N_UP - N_U, L_IMP), jnp.int32)])
    impg = imp_p.reshape(NC * NS, 125, 128)
    impu = jnp.repeat(jnp.arange(N_UP, dtype=jnp.int32) % 5120,
                      L_IMP).reshape(NC * NS, 125, 128)
    yt = jnp.pad(Y_table.at[0].set(0.0), ((0, 0), (0, 64)))

    degu_p, degi_p, y_pack = _sc_pre(degg, degs, onest, impg, impu, yt)
    deg_u = degu_p.reshape(1280, 8, 16)[:, :, 0].reshape(N_UP)[:N_U]
    deg_i = degi_p.reshape(128, 8, 16)[:, :, 0].reshape(1024)[:N_I]
    y_acc = y_pack[:, :D_OUT]
    r_u = 1.0 / jnp.sqrt(jnp.maximum(deg_u, 1.0))
    r_i = 1.0 / jnp.sqrt(jnp.maximum(deg_i, 1.0))

    ufeat_s = ufeat * r_u[:, None]
    ufs2 = ufeat_s.reshape(2 * N_U, 128)
    msgs = jnp.einsum('rio,ni->rno', W_i, ifeat)
    msgs_s = (msgs * r_i[None, :, None]).reshape(N_R * N_I, D_AGG)
    msA = msgs_s[:, :128]
    msB = msgs_s[:, 128:]

    a_rows, s_rows = _sc_edge(ufs2, msA, msB, g0, s0, g1, s1)
    A = a_rows.reshape(5120, 256)[:N_R * N_I].reshape(N_R, N_I, D_AGG)
    S = jnp.concatenate([s_rows[:N_U], s_rows[N_UP:N_UP + N_U]], axis=1)

    hi = r_i[:, None] * jnp.einsum('rdi,rio->do', A, W_u)
    hu = r_u[:, None] * S
    act = lambda t: jnp.where(t >= 0, t, 0.1 * t)
    p_mu = act(hu) @ W_uo + b_uo
    q_mu = act(hi) @ W_io + b_io
    y_mu = y_acc[:N_U] / sqrt_counts
    return q_mu @ (p_mu + y_mu).T + Bi + Bu.T + global_mean
